# Initial kernel scaffold; baseline (speedup 1.0000x reference)
import jax
import jax.numpy as jnp
from jax.experimental import pallas as pl


def kernel(x, batch, Wm, bm, Wx, bx, Ws, bs, Wgm, bgm, Wgx, bgx, Wgs, bgs, Wo, bo, gamma, beta):
    raise NotImplementedError("write your pallas kernel here")


# speedup vs baseline: 4.1866x; 4.1866x over previous
import jax
import jax.numpy as jnp
from jax.experimental import pallas as pl
from jax.experimental.pallas import tpu as pltpu

N = 100000
D = 128
S = 256
EPS = 1e-5
BLK = 1000
GRID = N // BLK

_NEG_INF = float("-inf")


def _dot_t(a, b):
    return jax.lax.dot_general(a, b, (((1,), (1,)), ((), ())),
                               preferred_element_type=jnp.float32)


def _pool_body(xb, segb, Wm, bm, Wx, bx, Ws, bs, Wgm, bgm, Wgx, bgx,
               Wgs, bgs, Wo, bo, gamma, beta, out_ref,
               sum_s, max_s, cnt_s):
    pi = pl.program_id(0)

    @pl.when(pi == 0)
    def _init():
        sum_s[...] = jnp.zeros_like(sum_s)
        max_s[...] = jnp.full_like(max_s, _NEG_INF)
        cnt_s[...] = jnp.zeros_like(cnt_s)

    seg = segb[...]
    xv = xb[...]
    smin = jnp.min(seg)
    smax = jnp.max(seg)

    def seg_iter(s, carry):
        m = seg == s
        contrib_sum = jnp.sum(jnp.where(m, xv, 0.0), axis=0, keepdims=True)
        contrib_max = jnp.max(jnp.where(m, xv, _NEG_INF), axis=0,
                              keepdims=True)
        contrib_cnt = jnp.sum(m.astype(jnp.float32))
        sum_s[pl.ds(s, 1), :] += contrib_sum
        max_s[pl.ds(s, 1), :] = jnp.maximum(max_s[pl.ds(s, 1), :],
                                            contrib_max)
        cnt_s[pl.ds(s, 1), :] += contrib_cnt
        return carry

    jax.lax.fori_loop(smin, smax + 1, seg_iter, 0)

    @pl.when(pi == GRID - 1)
    def _tail():
        sum_pool = sum_s[...]
        max_pool = max_s[...]
        counts = jnp.maximum(cnt_s[...], 1.0)
        mean_pool = sum_pool / counts

        mean_repr = _dot_t(mean_pool, Wm[...]) + bm[...]
        max_repr = _dot_t(max_pool, Wx[...]) + bx[...]
        sum_repr = _dot_t(sum_pool, Ws[...]) + bs[...]

        gm = _dot_t(mean_repr, Wgm[...]) + bgm[...]
        gx = _dot_t(max_repr, Wgx[...]) + bgx[...]
        gs = _dot_t(sum_repr, Wgs[...]) + bgs[...]
        gm = 1.0 / (1.0 + jnp.exp(-gm))
        gx = 1.0 / (1.0 + jnp.exp(-gx))
        gs = 1.0 / (1.0 + jnp.exp(-gs))

        mx = jnp.maximum(jnp.maximum(gm, gx), gs)
        em = jnp.exp(gm - mx)
        ex = jnp.exp(gx - mx)
        es = jnp.exp(gs - mx)
        denom = em + ex + es
        pooled = (em * mean_repr + ex * max_repr + es * sum_repr) / denom

        ge = _dot_t(pooled, Wo[...]) + bo[...]
        ones = jnp.ones((D, D), dtype=jnp.float32)
        mu = _dot_t(ge, ones) * (1.0 / D)
        dev = ge - mu
        var = _dot_t(dev * dev, ones) * (1.0 / D)
        out_ref[...] = dev / jnp.sqrt(var + EPS) * gamma[...] + beta[...]


def kernel(x, batch, Wm, bm, Wx, bx, Ws, bs, Wgm, bgm, Wgx, bgx,
           Wgs, bgs, Wo, bo, gamma, beta):
    seg = batch.astype(jnp.int32).reshape(N, 1)
    full = lambda i: (0, 0)
    wspec = lambda a: pl.BlockSpec(a.shape, full)
    b2 = lambda b: b.reshape(1, -1)

    bm2, bx2, bs2 = b2(bm), b2(bx), b2(bs)
    bo2, gamma2, beta2 = b2(bo), b2(gamma), b2(beta)
    Wgm_r = jnp.broadcast_to(Wgm, (D, D))
    Wgx_r = jnp.broadcast_to(Wgx, (D, D))
    Wgs_r = jnp.broadcast_to(Wgs, (D, D))
    bgm_r = jnp.broadcast_to(bgm.reshape(1, 1), (1, D))
    bgx_r = jnp.broadcast_to(bgx.reshape(1, 1), (1, D))
    bgs_r = jnp.broadcast_to(bgs.reshape(1, 1), (1, D))

    args = (x, seg, Wm, bm2, Wx, bx2, Ws, bs2, Wgm_r, bgm_r, Wgx_r, bgx_r,
            Wgs_r, bgs_r, Wo, bo2, gamma2, beta2)
    in_specs = [
        pl.BlockSpec((BLK, D), lambda i: (i, 0)),
        pl.BlockSpec((BLK, 1), lambda i: (i, 0)),
    ] + [wspec(a) for a in args[2:]]

    return pl.pallas_call(
        _pool_body,
        grid=(GRID,),
        in_specs=in_specs,
        out_specs=pl.BlockSpec((S, D), full),
        out_shape=jax.ShapeDtypeStruct((S, D), jnp.float32),
        scratch_shapes=[
            pltpu.VMEM((S, D), jnp.float32),
            pltpu.VMEM((S, D), jnp.float32),
            pltpu.VMEM((S, D), jnp.float32),
        ],
        compiler_params=pltpu.CompilerParams(
            dimension_semantics=("arbitrary",),
        ),
    )(*args)
